# baseline (speedup 1.0000x reference)
import functools

import jax
import jax.numpy as jnp
from jax import lax
from jax.experimental import pallas as pl
from jax.experimental.pallas import tpu as pltpu
from jax.experimental.pallas import tpu_sc as plsc

EMBED_DIM = 64
NUM_CORES = 2
NUM_SUBCORES = 16
NUM_WORKERS = NUM_CORES * NUM_SUBCORES
CHUNK = 128

_mesh = plsc.VectorSubcoreMesh(
    core_axis_name="c",
    subcore_axis_name="s",
    num_cores=NUM_CORES,
    num_subcores=NUM_SUBCORES,
)


@functools.partial(jax.jit, static_argnames=("n_chunks",))
def _lookup(ids, table, n_chunks):
    @functools.partial(
        pl.kernel,
        out_type=jax.ShapeDtypeStruct(
            (NUM_WORKERS, n_chunks, CHUNK, EMBED_DIM), jnp.float32
        ),
        mesh=_mesh,
        compiler_params=pltpu.CompilerParams(use_tc_tiling_on_sc=False),
        scratch_types=[
            pltpu.VMEM((n_chunks, CHUNK), jnp.int32),
            pltpu.VMEM((CHUNK, EMBED_DIM), jnp.float32),
            pltpu.VMEM((CHUNK, EMBED_DIM), jnp.float32),
            pltpu.SemaphoreType.DMA,
            pltpu.SemaphoreType.DMA,
        ],
    )
    def body(ids_hbm, table_hbm, out_hbm, idx_v, buf0, buf1, sem0, sem1):
        wid = lax.axis_index("s") * NUM_CORES + lax.axis_index("c")
        pltpu.sync_copy(ids_hbm.at[wid], idx_v)

        bufs = (buf0, buf1)
        sems = (sem0, sem1)

        pltpu.async_copy(table_hbm.at[idx_v.at[0]], buf0, sem0)
        pltpu.async_copy(table_hbm.at[idx_v.at[1]], buf1, sem1)

        @pl.loop(0, n_chunks - 2, step=2)
        def _(g):
            for b in range(2):
                j = g + b
                pltpu.make_async_copy(
                    table_hbm.at[idx_v.at[j]], bufs[b], sems[b]
                ).wait()
                pltpu.sync_copy(bufs[b], out_hbm.at[wid, j])
                pltpu.async_copy(table_hbm.at[idx_v.at[j + 2]], bufs[b], sems[b])

        for b in range(2):
            j = n_chunks - 2 + b
            pltpu.make_async_copy(
                table_hbm.at[idx_v.at[j]], bufs[b], sems[b]
            ).wait()
            pltpu.sync_copy(bufs[b], out_hbm.at[wid, j])

    return body(ids, table)


def kernel(token_ids, table):
    batch, seq = token_ids.shape
    total = batch * seq
    assert total % (NUM_WORKERS * CHUNK) == 0
    n_chunks = total // (NUM_WORKERS * CHUNK)
    ids = token_ids.reshape(NUM_WORKERS, n_chunks, CHUNK).astype(jnp.int32)
    out = _lookup(ids, table, n_chunks)
    return out.reshape(batch, seq, EMBED_DIM)


# speedup vs baseline: 1.1632x; 1.1632x over previous
import functools
import math

import jax
import jax.numpy as jnp
from jax import lax
from jax.experimental import pallas as pl
from jax.experimental.pallas import tpu as pltpu
from jax.experimental.pallas import tpu_sc as plsc

EMBED_DIM = 64
PADDED_DIM = 128
NUM_CORES = 2
NUM_SUBCORES = 16
NUM_WORKERS = NUM_CORES * NUM_SUBCORES
CHUNK = 128
TBLK = 2048

_mesh = plsc.VectorSubcoreMesh(
    core_axis_name="c",
    subcore_axis_name="s",
    num_cores=NUM_CORES,
    num_subcores=NUM_SUBCORES,
)


def _tc_stage(tab_t):
    d, v = tab_t.shape

    def body(in_ref, out_ref):
        out_ref[:, 0:EMBED_DIM] = in_ref[...].T

    return pl.pallas_call(
        body,
        grid=(math.ceil(v / TBLK),),
        in_specs=[pl.BlockSpec((d, TBLK), lambda g: (0, g))],
        out_specs=pl.BlockSpec((TBLK, PADDED_DIM), lambda g: (g, 0)),
        out_shape=jax.ShapeDtypeStruct((v, PADDED_DIM), jnp.float32),
    )(tab_t)


@functools.partial(jax.jit, static_argnames=("n_chunks",))
def _lookup(ids, table_p, n_chunks):
    @functools.partial(
        pl.kernel,
        out_type=jax.ShapeDtypeStruct(
            (NUM_WORKERS, n_chunks, CHUNK, EMBED_DIM), jnp.float32
        ),
        mesh=_mesh,
        compiler_params=pltpu.CompilerParams(use_tc_tiling_on_sc=False),
        scratch_types=[
            pltpu.VMEM((n_chunks, CHUNK), jnp.int32),
            pltpu.VMEM((CHUNK, PADDED_DIM), jnp.float32),
            pltpu.VMEM((CHUNK, PADDED_DIM), jnp.float32),
            pltpu.SemaphoreType.DMA,
            pltpu.SemaphoreType.DMA,
        ],
    )
    def body(ids_hbm, table_hbm, out_hbm, idx_v, buf0, buf1, sem0, sem1):
        wid = lax.axis_index("s") * NUM_CORES + lax.axis_index("c")
        pltpu.sync_copy(ids_hbm.at[wid], idx_v)

        bufs = (buf0, buf1)
        sems = (sem0, sem1)

        pltpu.async_copy(table_hbm.at[idx_v.at[0]], buf0, sem0)
        pltpu.async_copy(table_hbm.at[idx_v.at[1]], buf1, sem1)

        @pl.loop(0, n_chunks - 2, step=2)
        def _(g):
            for b in range(2):
                j = g + b
                pltpu.make_async_copy(
                    table_hbm.at[idx_v.at[j]], bufs[b], sems[b]
                ).wait()
                pltpu.sync_copy(
                    bufs[b].at[:, pl.ds(0, EMBED_DIM)], out_hbm.at[wid, j]
                )
                pltpu.async_copy(table_hbm.at[idx_v.at[j + 2]], bufs[b], sems[b])

        for b in range(2):
            j = n_chunks - 2 + b
            pltpu.make_async_copy(
                table_hbm.at[idx_v.at[j]], bufs[b], sems[b]
            ).wait()
            pltpu.sync_copy(
                bufs[b].at[:, pl.ds(0, EMBED_DIM)], out_hbm.at[wid, j]
            )

    return body(ids, table_p)


def kernel(token_ids, table):
    batch, seq = token_ids.shape
    total = batch * seq
    assert total % (NUM_WORKERS * CHUNK) == 0
    n_chunks = total // (NUM_WORKERS * CHUNK)
    ids = token_ids.reshape(NUM_WORKERS, n_chunks, CHUNK).astype(jnp.int32)
    table_p = _tc_stage(table.T)
    out = _lookup(ids, table_p, n_chunks)
    return out.reshape(batch, seq, EMBED_DIM)
